# Initial kernel scaffold; baseline (speedup 1.0000x reference)
import jax
import jax.numpy as jnp
from jax import lax
from jax.experimental import pallas as pl
from jax.experimental.pallas import tpu as pltpu
from jax.experimental.pallas import tpu_sc as plsc

N = 10000
E = 320000
D = 128
NC = 2
NS = 16
NW = NC * NS
EPT = E // NW
C = 80
RPT = N // NS
DEG_W = 16

_mesh = plsc.VectorSubcoreMesh(core_axis_name="c", subcore_axis_name="s")


def _degree_body(src_hbm, dst_hbm, ones_hbm, zeros_hbm, deg_o_hbm, deg_i_hbm,
                 sidx, didx, ones_v, acc_o, acc_i):
    cid = lax.axis_index("c")
    sid = lax.axis_index("s")
    base = (cid * NS + sid) * EPT
    row0 = sid * RPT
    pltpu.sync_copy(zeros_hbm.at[pl.ds(row0, RPT)], acc_o.at[pl.ds(row0, RPT)])
    pltpu.sync_copy(zeros_hbm.at[pl.ds(row0, RPT)], acc_i.at[pl.ds(row0, RPT)])
    pltpu.sync_copy(ones_hbm, ones_v)
    plsc.subcore_barrier()

    @pl.loop(0, EPT, step=C)
    def _(i):
        off = base + i
        pltpu.sync_copy(src_hbm.at[pl.ds(off, C)], sidx)
        pltpu.sync_copy(dst_hbm.at[pl.ds(off, C)], didx)
        pltpu.sync_copy(ones_v, acc_o.at[sidx], add=True)
        pltpu.sync_copy(ones_v, acc_i.at[didx], add=True)

    plsc.subcore_barrier()
    pltpu.sync_copy(acc_o.at[pl.ds(row0, RPT)], deg_o_hbm.at[cid, pl.ds(row0, RPT)])
    pltpu.sync_copy(acc_i.at[pl.ds(row0, RPT)], deg_i_hbm.at[cid, pl.ds(row0, RPT)])


_deg_call = pl.kernel(
    _degree_body,
    out_type=(jax.ShapeDtypeStruct((NC, N, DEG_W), jnp.float32),
              jax.ShapeDtypeStruct((NC, N, DEG_W), jnp.float32)),
    mesh=_mesh,
    scratch_types=[
        pltpu.VMEM((C,), jnp.int32),
        pltpu.VMEM((C,), jnp.int32),
        pltpu.VMEM((C, DEG_W), jnp.float32),
        pltpu.VMEM_SHARED((N, DEG_W), jnp.float32),
        pltpu.VMEM_SHARED((N, DEG_W), jnp.float32),
    ],
)


def _agg_body(x_hbm, src_hbm, dst_hbm, zeros_hbm, out_hbm, sidx, didx, rows, acc):
    cid = lax.axis_index("c")
    sid = lax.axis_index("s")
    base = (cid * NS + sid) * EPT
    row0 = sid * RPT
    pltpu.sync_copy(zeros_hbm.at[pl.ds(row0, RPT)], acc.at[pl.ds(row0, RPT)])
    plsc.subcore_barrier()

    @pl.loop(0, EPT, step=C)
    def _(i):
        off = base + i
        pltpu.sync_copy(src_hbm.at[pl.ds(off, C)], sidx)
        pltpu.sync_copy(dst_hbm.at[pl.ds(off, C)], didx)
        pltpu.sync_copy(x_hbm.at[sidx], rows)
        pltpu.sync_copy(rows, acc.at[didx], add=True)

    plsc.subcore_barrier()
    pltpu.sync_copy(acc.at[pl.ds(row0, RPT)], out_hbm.at[cid, pl.ds(row0, RPT)])


_agg_call = pl.kernel(
    _agg_body,
    out_type=jax.ShapeDtypeStruct((NC, N, D), jnp.float32),
    mesh=_mesh,
    scratch_types=[
        pltpu.VMEM((C,), jnp.int32),
        pltpu.VMEM((C,), jnp.int32),
        pltpu.VMEM((C, D), jnp.float32),
        pltpu.VMEM_SHARED((N, D), jnp.float32),
    ],
)


def _prep_body(feat_ref, deg_o_ref, deg_i_ref, xs_ref, ns_ref, nd_ref):
    d_o = deg_o_ref[0, :, 0:1] + deg_o_ref[1, :, 0:1]
    d_i = deg_i_ref[0, :, 0:1] + deg_i_ref[1, :, 0:1]
    ns = jnp.where(d_o > 0, lax.rsqrt(d_o), 0.0)
    nd = jnp.where(d_i > 0, lax.rsqrt(d_i), 0.0)
    ns_ref[...] = ns
    nd_ref[...] = nd
    xs_ref[...] = feat_ref[...] * ns


_prep_call = pl.pallas_call(
    _prep_body,
    out_shape=(jax.ShapeDtypeStruct((N, D), jnp.float32),
               jax.ShapeDtypeStruct((N, 1), jnp.float32),
               jax.ShapeDtypeStruct((N, 1), jnp.float32)),
)


def _layer1_body(p_ref, nd_ref, ns_ref, w_ref, b_ref, o_ref):
    t = (p_ref[0] + p_ref[1]) * nd_ref[...]
    h = jnp.dot(t, w_ref[...], preferred_element_type=jnp.float32) + b_ref[...]
    o_ref[...] = jnp.maximum(h, 0.0) * ns_ref[...]


_layer1_call = pl.pallas_call(
    _layer1_body,
    out_shape=jax.ShapeDtypeStruct((N, D), jnp.float32),
)


def _layer2_body(p_ref, nd_ref, w_ref, b_ref, o_ref):
    t = (p_ref[0] + p_ref[1]) * nd_ref[...]
    o_ref[...] = jnp.dot(t, w_ref[...], preferred_element_type=jnp.float32) + b_ref[...]


_layer2_call = pl.pallas_call(
    _layer2_body,
    out_shape=jax.ShapeDtypeStruct((N, D), jnp.float32),
)


def kernel(feat, edge_index, W1, b1, W2, b2):
    src = edge_index[0].astype(jnp.int32)
    dst = edge_index[1].astype(jnp.int32)
    zeros_nd = jnp.zeros((N, D), jnp.float32)
    zeros_n16 = jnp.zeros((N, DEG_W), jnp.float32)
    ones_c16 = jnp.ones((C, DEG_W), jnp.float32)

    deg_o, deg_i = _deg_call(src, dst, ones_c16, zeros_n16)
    xs, ns, nd = _prep_call(feat, deg_o, deg_i)
    p1 = _agg_call(xs, src, dst, zeros_nd)
    x2 = _layer1_call(p1, nd, ns, W1, b1.reshape(1, D))
    p2 = _agg_call(x2, src, dst, zeros_nd)
    return _layer2_call(p2, nd, W2, b2.reshape(1, D))


# speedup vs baseline: 4.4268x; 4.4268x over previous
import jax
import jax.numpy as jnp
from jax import lax
from jax.experimental import pallas as pl
from jax.experimental.pallas import tpu as pltpu
from jax.experimental.pallas import tpu_sc as plsc

N = 10000
NPAD = 10240
E = 320000
D = 128
NC = 2
NS = 16
NW = NC * NS
EPT = E // NW
C = 80
EPC = E // NS
RPT = NPAD // NS

_mesh = plsc.VectorSubcoreMesh(core_axis_name="c", subcore_axis_name="s",
                               num_cores=NC, num_subcores=NS)


def _degree_body(edges_hbm, zeros_hbm, deg_hbm,
                 sidx, ones_v, acc):
    cid = lax.axis_index("c")
    sid = lax.axis_index("s")
    base = cid * E + sid * EPC
    row0 = sid * RPT
    pltpu.sync_copy(zeros_hbm.at[pl.ds(row0, RPT)], acc.at[pl.ds(row0, RPT)])
    ones = jnp.full((16,), 1.0, dtype=jnp.float32)

    @pl.loop(0, C)
    def _(r):
        for k in range(D // 16):
            ones_v[r, pl.ds(k * 16, 16)] = ones

    plsc.subcore_barrier()

    @pl.loop(0, EPC, step=C)
    def _(i):
        pltpu.sync_copy(edges_hbm.at[pl.ds(base + i, C)], sidx)
        pltpu.sync_copy(ones_v, acc.at[sidx], add=True)

    plsc.subcore_barrier()
    pltpu.sync_copy(acc.at[pl.ds(row0, RPT)], deg_hbm.at[cid, pl.ds(row0, RPT)])


_deg_call = pl.kernel(
    _degree_body,
    out_type=jax.ShapeDtypeStruct((NC, NPAD, D), jnp.float32),
    mesh=_mesh,
    scratch_types=[
        pltpu.VMEM((C,), jnp.int32),
        pltpu.VMEM((C, D), jnp.float32),
        pltpu.VMEM_SHARED((NPAD, D), jnp.float32),
    ],
)


def _agg_body(x_hbm, src_hbm, dst_hbm, zeros_hbm, out_hbm, sidx, didx, rows, acc):
    cid = lax.axis_index("c")
    sid = lax.axis_index("s")
    base = (cid * NS + sid) * EPT
    row0 = sid * RPT
    pltpu.sync_copy(zeros_hbm.at[pl.ds(row0, RPT)], acc.at[pl.ds(row0, RPT)])
    plsc.subcore_barrier()

    @pl.loop(0, EPT, step=C)
    def _(i):
        off = base + i
        pltpu.sync_copy(src_hbm.at[pl.ds(off, C)], sidx)
        pltpu.sync_copy(dst_hbm.at[pl.ds(off, C)], didx)
        pltpu.sync_copy(x_hbm.at[sidx], rows)
        pltpu.sync_copy(rows, acc.at[didx], add=True)

    plsc.subcore_barrier()
    pltpu.sync_copy(acc.at[pl.ds(row0, RPT)], out_hbm.at[cid, pl.ds(row0, RPT)])


_agg_call = pl.kernel(
    _agg_body,
    out_type=jax.ShapeDtypeStruct((NC, NPAD, D), jnp.float32),
    mesh=_mesh,
    scratch_types=[
        pltpu.VMEM((C,), jnp.int32),
        pltpu.VMEM((C,), jnp.int32),
        pltpu.VMEM((C, D), jnp.float32),
        pltpu.VMEM_SHARED((NPAD, D), jnp.float32),
    ],
)


def _prep_body(feat_ref, deg_ref, xs_ref, ns_ref, nd_ref):
    d_o = deg_ref[0, :N, 0:1]
    d_i = deg_ref[1, :N, 0:1]
    ns = jnp.where(d_o > 0, lax.rsqrt(d_o), 0.0)
    nd = jnp.where(d_i > 0, lax.rsqrt(d_i), 0.0)
    ns_ref[...] = ns
    nd_ref[...] = nd
    xs_ref[...] = feat_ref[...] * ns


_prep_call = pl.pallas_call(
    _prep_body,
    out_shape=(jax.ShapeDtypeStruct((N, D), jnp.float32),
               jax.ShapeDtypeStruct((N, 1), jnp.float32),
               jax.ShapeDtypeStruct((N, 1), jnp.float32)),
)


def _layer1_body(p_ref, nd_ref, ns_ref, w_ref, b_ref, o_ref):
    t = (p_ref[0, :N] + p_ref[1, :N]) * nd_ref[...]
    h = jnp.dot(t, w_ref[...], preferred_element_type=jnp.float32) + b_ref[...]
    o_ref[...] = jnp.maximum(h, 0.0) * ns_ref[...]


_layer1_call = pl.pallas_call(
    _layer1_body,
    out_shape=jax.ShapeDtypeStruct((N, D), jnp.float32),
)


def _layer2_body(p_ref, nd_ref, w_ref, b_ref, o_ref):
    t = (p_ref[0, :N] + p_ref[1, :N]) * nd_ref[...]
    o_ref[...] = jnp.dot(t, w_ref[...], preferred_element_type=jnp.float32) + b_ref[...]


_layer2_call = pl.pallas_call(
    _layer2_body,
    out_shape=jax.ShapeDtypeStruct((N, D), jnp.float32),
)


def kernel(feat, edge_index, W1, b1, W2, b2):
    src = edge_index[0].astype(jnp.int32)
    dst = edge_index[1].astype(jnp.int32)
    zeros_nd = jnp.zeros((NPAD, D), jnp.float32)

    deg = _deg_call(edge_index.reshape(-1).astype(jnp.int32), zeros_nd)
    xs, ns, nd = _prep_call(feat, deg)
    p1 = _agg_call(xs, src, dst, zeros_nd)
    x2 = _layer1_call(p1, nd, ns, W1, b1.reshape(1, D))
    p2 = _agg_call(x2, src, dst, zeros_nd)
    return _layer2_call(p2, nd, W2, b2.reshape(1, D))
